# baseline (speedup 1.0000x reference)
import functools

import jax
import jax.numpy as jnp
from jax import lax
from jax.experimental import pallas as pl
from jax.experimental.pallas import tpu as pltpu
from jax.experimental.pallas import tpu_sc as plsc

_DIM = 768
_NUM_EXPERTS = 64
_TOP_K = 2
_CAPACITY_FACTOR = 1.0
_SEQ = 32768

_NC = 2
_NS = 16
_NW = _NC * _NS
_TPW = _SEQ // _NW


def _router_body(x_ref, w_ref, b_ref, logits_ref, idx_ref, wts_ref):
    x = x_ref[...]
    w = w_ref[...]
    logits = jax.lax.dot_general(
        x, w, dimension_numbers=(((1,), (1,)), ((), ())),
        preferred_element_type=jnp.float32,
    ) + b_ref[...]
    logits_ref[...] = logits

    e = jax.lax.broadcasted_iota(jnp.int32, logits.shape, 1)
    big = jnp.int32(_NUM_EXPERTS)

    m1 = jnp.max(logits, axis=1, keepdims=True)
    i1 = jnp.min(jnp.where(logits == m1, e, big), axis=1, keepdims=True)
    masked = jnp.where(e == i1, -jnp.inf, logits)
    m2 = jnp.max(masked, axis=1, keepdims=True)
    i2 = jnp.min(jnp.where(masked == m2, e, big), axis=1, keepdims=True)

    idx_ref[...] = jnp.concatenate([i1, i2], axis=1)

    w1 = 1.0 / (1.0 + jnp.exp(m2 - m1))
    wts_ref[...] = jnp.concatenate([w1, 1.0 - w1], axis=1)


def _mask_body(idx_hbm, mask_hbm, idx_v, mask_v):
    wid = lax.axis_index("s") * _NC + lax.axis_index("c")
    pltpu.sync_copy(idx_hbm.at[pl.ds(wid * _TPW * _TOP_K, _TPW * _TOP_K)], idx_v)

    iota16 = lax.iota(jnp.int32, 16)
    one16 = jnp.ones((16,), jnp.float32)
    zero16 = jnp.zeros((16,), jnp.float32)

    def group(g, carry):
        vals = idx_v[pl.ds(g * 16, 16)]
        for j in range(8):
            b1 = jnp.full((16,), vals[2 * j], jnp.int32)
            b2 = jnp.full((16,), vals[2 * j + 1], jnp.int32)
            t = g * 8 + j
            for c in range(_NUM_EXPERTS // 16):
                lane = iota16 + c * 16
                vec = jnp.where((lane == b1) | (lane == b2), one16, zero16)
                mask_v[pl.ds(t * _NUM_EXPERTS + c * 16, 16)] = vec
        return carry

    lax.fori_loop(0, _TPW * _TOP_K // 16, group, 0)

    pltpu.sync_copy(
        mask_v,
        mask_hbm.at[pl.ds(wid * _TPW * _NUM_EXPERTS, _TPW * _NUM_EXPERTS)],
    )


_mask_kernel = functools.partial(
    pl.kernel,
    out_type=jax.ShapeDtypeStruct((_SEQ * _NUM_EXPERTS,), jnp.float32),
    mesh=plsc.VectorSubcoreMesh(core_axis_name="c", subcore_axis_name="s"),
    scratch_types=[
        pltpu.VMEM((_TPW * _TOP_K,), jnp.int32),
        pltpu.VMEM((_TPW * _NUM_EXPERTS,), jnp.float32),
    ],
)(_mask_body)


@jax.jit
def kernel(x, W, b):
    seq_len, dim = x.shape
    num_experts = W.shape[0]
    block = 4096
    grid = (seq_len // block,)

    b2 = b.reshape(1, num_experts)

    out_shapes = (
        jax.ShapeDtypeStruct((seq_len, num_experts), jnp.float32),
        jax.ShapeDtypeStruct((seq_len, _TOP_K), jnp.int32),
        jax.ShapeDtypeStruct((seq_len, _TOP_K), jnp.float32),
    )

    logits, idx, wts = pl.pallas_call(
        _router_body,
        grid=grid,
        in_specs=[
            pl.BlockSpec((block, dim), lambda i: (i, 0)),
            pl.BlockSpec((num_experts, dim), lambda i: (0, 0)),
            pl.BlockSpec((1, num_experts), lambda i: (0, 0)),
        ],
        out_specs=(
            pl.BlockSpec((block, num_experts), lambda i: (i, 0)),
            pl.BlockSpec((block, _TOP_K), lambda i: (i, 0)),
            pl.BlockSpec((block, _TOP_K), lambda i: (i, 0)),
        ),
        out_shape=out_shapes,
        compiler_params=pltpu.CompilerParams(
            dimension_semantics=("parallel",),
        ),
    )(x, W, b2)

    mask = _mask_kernel(idx.reshape(-1)).reshape(seq_len, num_experts)

    capacity = jnp.int32(
        min(seq_len, int(_CAPACITY_FACTOR * seq_len / num_experts * _TOP_K))
    )
    return (logits, idx, wts, mask, capacity)


# speedup vs baseline: 1.3030x; 1.3030x over previous
import jax
import jax.numpy as jnp
from jax.experimental import pallas as pl
from jax.experimental.pallas import tpu as pltpu

_DIM = 768
_NUM_EXPERTS = 64
_TOP_K = 2
_CAPACITY_FACTOR = 1.0


def _router_body(x_ref, w_ref, b_ref, logits_ref, idx_ref, wts_ref, mask_ref):
    i = pl.program_id(0)
    block = x_ref.shape[0]
    rows = pl.ds(i * block, block)
    x = x_ref[...]
    w = w_ref[...]
    logits = jax.lax.dot_general(
        x, w, dimension_numbers=(((1,), (1,)), ((), ())),
        preferred_element_type=jnp.float32,
    ) + b_ref[...]
    logits_ref[rows, :] = logits

    e = jax.lax.broadcasted_iota(jnp.int32, logits.shape, 1)
    big = jnp.int32(_NUM_EXPERTS)

    m1 = jnp.max(logits, axis=1, keepdims=True)
    i1 = jnp.min(jnp.where(logits == m1, e, big), axis=1, keepdims=True)
    masked = jnp.where(e == i1, -jnp.inf, logits)
    m2 = jnp.max(masked, axis=1, keepdims=True)
    i2 = jnp.min(jnp.where(masked == m2, e, big), axis=1, keepdims=True)

    idx_ref[...] = jnp.concatenate([i1, i2], axis=1)

    w1 = 1.0 / (1.0 + jnp.exp(m2 - m1))
    wts_ref[...] = jnp.concatenate([w1, 1.0 - w1], axis=1)

    mask_ref[rows, :] = ((e == i1) | (e == i2)).astype(jnp.float32)


@jax.jit
def kernel(x, W, b):
    seq_len, dim = x.shape
    num_experts = W.shape[0]
    block = 2048
    grid = (seq_len // block,)

    b2 = b.reshape(1, num_experts)

    out_shapes = (
        jax.ShapeDtypeStruct((seq_len, num_experts), jnp.float32),
        jax.ShapeDtypeStruct((seq_len, _TOP_K), jnp.int32),
        jax.ShapeDtypeStruct((seq_len, _TOP_K), jnp.float32),
        jax.ShapeDtypeStruct((seq_len, num_experts), jnp.float32),
    )

    logits, idx, wts, mask = pl.pallas_call(
        _router_body,
        grid=grid,
        in_specs=[
            pl.BlockSpec((block, dim), lambda i: (i, 0)),
            pl.BlockSpec((num_experts, dim), lambda i: (0, 0)),
            pl.BlockSpec((1, num_experts), lambda i: (0, 0)),
        ],
        out_specs=(
            pl.BlockSpec((seq_len, num_experts), lambda i: (0, 0)),
            pl.BlockSpec((block, _TOP_K), lambda i: (i, 0)),
            pl.BlockSpec((block, _TOP_K), lambda i: (i, 0)),
            pl.BlockSpec((seq_len, num_experts), lambda i: (0, 0)),
        ),
        out_shape=out_shapes,
        compiler_params=pltpu.CompilerParams(
            dimension_semantics=("arbitrary",),
        ),
    )(x, W, b2)

    capacity = jnp.int32(
        min(seq_len, int(_CAPACITY_FACTOR * seq_len / num_experts * _TOP_K))
    )
    return (logits, idx, wts, mask, capacity)


# speedup vs baseline: 1.3649x; 1.0475x over previous
import jax
import jax.numpy as jnp
from jax.experimental import pallas as pl
from jax.experimental.pallas import tpu as pltpu

_DIM = 768
_NUM_EXPERTS = 64
_TOP_K = 2
_CAPACITY_FACTOR = 1.0


def _router_body(x_ref, w_ref, b_ref, logits_ref, idx_ref, wts_ref, mask_ref):
    x = x_ref[...]
    w = w_ref[...]
    logits = jax.lax.dot_general(
        x, w, dimension_numbers=(((1,), (1,)), ((), ())),
        preferred_element_type=jnp.float32,
    ) + b_ref[...]
    logits_ref[...] = logits

    e = jax.lax.broadcasted_iota(jnp.int32, logits.shape, 1)
    big = jnp.int32(_NUM_EXPERTS)

    m1 = jnp.max(logits, axis=1, keepdims=True)
    i1 = jnp.min(jnp.where(logits == m1, e, big), axis=1, keepdims=True)
    masked = jnp.where(e == i1, -jnp.inf, logits)
    m2 = jnp.max(masked, axis=1, keepdims=True)
    i2 = jnp.min(jnp.where(masked == m2, e, big), axis=1, keepdims=True)

    idx_ref[...] = jnp.concatenate([i1, i2], axis=1)

    w1 = 1.0 / (1.0 + jnp.exp(m2 - m1))
    wts_ref[...] = jnp.concatenate([w1, 1.0 - w1], axis=1)

    mask_ref[...] = ((e == i1) | (e == i2)).astype(jnp.float32)


@jax.jit
def kernel(x, W, b):
    seq_len, dim = x.shape
    num_experts = W.shape[0]
    block = 4096
    grid = (seq_len // block,)

    b2 = b.reshape(1, num_experts)

    out_shapes = (
        jax.ShapeDtypeStruct((seq_len, num_experts), jnp.float32),
        jax.ShapeDtypeStruct((seq_len, _TOP_K), jnp.int32),
        jax.ShapeDtypeStruct((seq_len, _TOP_K), jnp.float32),
        jax.ShapeDtypeStruct((seq_len, num_experts), jnp.float32),
    )

    logits, idx, wts, mask = pl.pallas_call(
        _router_body,
        grid=grid,
        in_specs=[
            pl.BlockSpec((block, dim), lambda i: (i, 0)),
            pl.BlockSpec((num_experts, dim), lambda i: (0, 0)),
            pl.BlockSpec((1, num_experts), lambda i: (0, 0)),
        ],
        out_specs=(
            pl.BlockSpec((block, num_experts), lambda i: (i, 0)),
            pl.BlockSpec((block, _TOP_K), lambda i: (i, 0)),
            pl.BlockSpec((block, _TOP_K), lambda i: (i, 0)),
            pl.BlockSpec((block, num_experts), lambda i: (i, 0)),
        ),
        out_shape=out_shapes,
        compiler_params=pltpu.CompilerParams(
            dimension_semantics=("parallel",),
        ),
    )(x, W, b2)

    capacity = jnp.int32(
        min(seq_len, int(_CAPACITY_FACTOR * seq_len / num_experts * _TOP_K))
    )
    return (logits, idx, wts, mask, capacity)
